# baseline (speedup 1.0000x reference)
import functools

import jax
import jax.numpy as jnp
from jax.experimental import pallas as pl
from jax.experimental.pallas import tpu as pltpu

_H = 128
_NG = 16
_CUT = 10.0
_NE = 4
_NM = 1024
_NF = 128
_BM = 128
_DELTA = _CUT / (_NG - 1)
_COEFF = -0.5 / (_DELTA * _DELTA)
_SCALE = (-_COEFF) ** 0.5
_HIGH = jax.lax.Precision.HIGHEST


def _edge_kernel(xf_ref, pf_ref, xm_ref, pm_ref, w1_ref, b1_ref, w2_ref,
                 b2_ref, offs_ref, ff_ref, mf_ref):
    pid = pl.program_id(0)
    is_ff = pid == 0
    nj = _NF
    e = _BM * nj

    x = jnp.where(is_ff, xf_ref[...], xm_ref[...])
    tsel = jnp.where(is_ff, 0.0, 1.0)
    a = (jnp.dot(x, w1_ref[0:_H, :], precision=_HIGH)
         + b1_ref[...] + tsel * w1_ref[2 * _H + _NG:2 * _H + _NG + 1, :])
    b = jnp.dot(xf_ref[...], w1_ref[_H:2 * _H, :], precision=_HIGH)

    px = jnp.where(is_ff, pf_ref[...], pm_ref[...]) * _SCALE
    qt = pf_ref[...].T * _SCALE
    d2 = ((px[:, 0:1] - qt[0:1, :]) ** 2
          + (px[:, 1:2] - qt[1:2, :]) ** 2
          + (px[:, 2:3] - qt[2:3, :]) ** 2)
    d = jnp.sqrt(d2 + (1e-12 * _SCALE * _SCALE))

    u = d[:, None, :] - offs_ref[...][None, :, :]
    s = jnp.exp(-(u * u))
    g3 = jax.lax.dot_general(
        s.astype(jnp.bfloat16),
        w1_ref[2 * _H:2 * _H + _NG, :].astype(jnp.bfloat16),
        dimension_numbers=(((1,), (0,)), ((), ())),
        preferred_element_type=jnp.float32)

    pre = g3 + a[:, None, :] + b[None, :, :]
    h = jnp.maximum(pre, 0.0).reshape(e, 2 * _H)
    ot = jax.lax.dot_general(
        w2_ref[...].astype(jnp.bfloat16), h.astype(jnp.bfloat16),
        dimension_numbers=(((0,), (1,)), ((), ())),
        preferred_element_type=jnp.float32) + b2_ref[...]

    @pl.when(is_ff)
    def _():
        ff_ref[...] = ot

    @pl.when(jnp.logical_not(is_ff))
    def _():
        mf_ref[...] = ot


@functools.partial(jax.jit, static_argnames=())
def kernel(h_mol, pos_mol, h_frag, pos_frag, batch_mol, batch_frag,
           W1, b1, W2, b2):
    full = lambda shape: pl.BlockSpec(shape, lambda i: (0,) * len(shape))
    mol_blk = lambda *tail: pl.BlockSpec(
        (_BM,) + tail, lambda i: (jnp.maximum(i - 1, 0),) + (0,) * len(tail))
    offs = (jnp.arange(_NG, dtype=jnp.float32) * (_DELTA * _SCALE))[:, None]
    dim_in = 2 * _H + _NG + 1
    e_blk = _BM * _NF
    fft, mft = pl.pallas_call(
        _edge_kernel,
        grid=(1 + _NM // _BM,),
        in_specs=[
            full((_NF, _H)),
            full((_NF, 3)),
            mol_blk(_H),
            mol_blk(3),
            full((dim_in, 2 * _H)),
            full((1, 2 * _H)),
            full((2 * _H, _NE)),
            full((_NE, 1)),
            full((_NG, 1)),
        ],
        out_specs=[
            pl.BlockSpec((_NE, _NF * _NF), lambda i: (0, 0)),
            pl.BlockSpec((_NE, e_blk), lambda i: (0, jnp.maximum(i - 1, 0))),
        ],
        out_shape=[
            jax.ShapeDtypeStruct((_NE, _NF * _NF), jnp.float32),
            jax.ShapeDtypeStruct((_NE, _NM * _NF), jnp.float32),
        ],
        compiler_params=pltpu.CompilerParams(
            dimension_semantics=("arbitrary",)),
    )(h_frag, pos_frag, h_mol, pos_mol, W1, b1[None, :], W2, b2[:, None], offs)
    ff = fft.T.reshape(_NF, _NF, _NE)
    mf = mft.T.reshape(_NM, _NF, _NE)
    return ff, mf
